# baseline (speedup 1.0000x reference)
import functools

import jax
import jax.numpy as jnp
from jax import lax
from jax.experimental import pallas as pl
from jax.experimental.pallas import tpu as pltpu
from jax.experimental.pallas import tpu_sc as plsc

NC = 2
NS = 16
NW = NC * NS
CE = 128


def _make_degree_kernel(ET, NA):
    mesh = plsc.VectorSubcoreMesh(core_axis_name="c", subcore_axis_name="s")

    @functools.partial(
        pl.kernel,
        out_type=jax.ShapeDtypeStruct((NW * NA,), jnp.float32),
        mesh=mesh,
        compiler_params=pltpu.CompilerParams(needs_layout_passes=False),
        scratch_types=[
            pltpu.VMEM((ET,), jnp.int32),
            pltpu.VMEM((NA,), jnp.float32),
        ],
    )
    def deg_kernel(idx_hbm, zeros_hbm, out_hbm, idx_v, local):
        cid = lax.axis_index("c")
        sid = lax.axis_index("s")
        wid = cid * NS + sid
        pltpu.sync_copy(zeros_hbm, local)
        pltpu.sync_copy(idx_hbm.at[pl.ds(wid * ET, ET)], idx_v)
        ones16 = jnp.ones((16,), jnp.float32)

        def body(i, carry):
            idx16 = idx_v[pl.ds(i * 16, 16)]
            plsc.addupdate_scatter(local, [idx16], ones16)
            return carry

        lax.fori_loop(0, ET // 16, body, 0)
        pltpu.sync_copy(local, out_hbm.at[pl.ds(wid * NA, NA)])

    return deg_kernel


def _make_edge_kernel(KM, NA, D):
    mesh = plsc.VectorSubcoreMesh(core_axis_name="c", subcore_axis_name="s")
    rows = NA // NS
    assert KM % 2 == 0

    @functools.partial(
        pl.kernel,
        out_type=jax.ShapeDtypeStruct((2, NA, D), jnp.float32),
        mesh=mesh,
        scratch_types=[
            [pltpu.VMEM((2, CE), jnp.int32)] * 2,
            [pltpu.VMEM((CE, D), jnp.float32)] * 2,
            [pltpu.SemaphoreType.DMA] * 2,
            [pltpu.SemaphoreType.DMA] * 2,
            pltpu.VMEM_SHARED((NA, D), jnp.float32),
        ],
    )
    def edge_kernel(pair_hbm, zs_hbm, zeros_hbm, out_hbm,
                    pair_c, bufs, psem, gsem, acc):
        cid = lax.axis_index("c")
        sid = lax.axis_index("s")
        wid = cid * NS + sid
        r0 = sid * rows
        e0 = wid * KM
        @pl.when(cid == 0)
        def _():
            pltpu.sync_copy(zs_hbm.at[pl.ds(r0, rows)], acc.at[pl.ds(r0, rows)])

        @pl.when(cid == 1)
        def _():
            pltpu.sync_copy(zeros_hbm.at[pl.ds(r0, rows)], acc.at[pl.ds(r0, rows)])

        for b in range(2):
            pltpu.async_copy(pair_hbm.at[e0 + b], pair_c[b], psem[b])
        plsc.subcore_barrier()
        pltpu.make_async_copy(pair_hbm.at[0], pair_c[0], psem[0]).wait()
        pltpu.async_copy(zs_hbm.at[pair_c[0].at[0]], bufs[0], gsem[0])

        def body(g, carry):
            for b in range(2):
                k = g * 2 + b
                nb = 1 - b

                @pl.when(k + 1 < KM)
                def _():
                    pltpu.make_async_copy(pair_hbm.at[0], pair_c[nb],
                                          psem[nb]).wait()
                    pltpu.async_copy(zs_hbm.at[pair_c[nb].at[0]], bufs[nb],
                                     gsem[nb])

                pltpu.make_async_copy(zs_hbm.at[pair_c[b].at[0]], bufs[b],
                                      gsem[b]).wait()
                pltpu.sync_copy(bufs[b], acc.at[pair_c[b].at[1]], add=True)

                @pl.when(k + 2 < KM)
                def _():
                    pltpu.async_copy(pair_hbm.at[e0 + k + 2], pair_c[b],
                                     psem[b])

            return carry

        lax.fori_loop(0, KM // 2, body, 0)
        plsc.subcore_barrier()
        pltpu.sync_copy(acc.at[pl.ds(r0, rows)], out_hbm.at[cid, pl.ds(r0, rows)])

    return edge_kernel


def _proj_body(next_h_ref, w_ref, tw_ref, out_ref):
    w2 = w_ref[...] * tw_ref[...]
    out_ref[...] = jnp.dot(next_h_ref[...], w2,
                           preferred_element_type=jnp.float32,
                           precision=lax.Precision.HIGHEST)


def _zs_body(inc_ref, p_ref, h_ref, wc_ref, cw_ref, deg_ref, out_ref, *, kb):
    k = pl.program_id(1)

    @pl.when(k == 0)
    def _():
        out_ref[...] = jnp.zeros_like(out_ref)

    out_ref[...] += jnp.dot(inc_ref[...], p_ref[...],
                            preferred_element_type=jnp.float32,
                            precision=lax.Precision.HIGHEST)

    @pl.when(k == kb - 1)
    def _():
        w1 = wc_ref[...] * cw_ref[...]
        z = out_ref[...] + jnp.dot(h_ref[...], w1,
                                   preferred_element_type=jnp.float32,
                                   precision=lax.Precision.HIGHEST)
        deg = jnp.sum(deg_ref[...], axis=-1, keepdims=True)
        out_ref[...] = z * lax.rsqrt(deg + 1.0)


def _final_body(p0_ref, p1_ref, deg_ref, bc_ref, cw_ref, bf_ref, tw_ref,
                g_ref, b_ref, out_ref):
    bias = bc_ref[...] * cw_ref[...] + bf_ref[...] * tw_ref[...]
    deg = jnp.sum(deg_ref[...], axis=-1, keepdims=True)
    x = (p0_ref[...] + p1_ref[...]) * lax.rsqrt(deg + 1.0) + bias
    mu = jnp.mean(x, axis=-1, keepdims=True)
    xc = x - mu
    var = jnp.mean(xc * xc, axis=-1, keepdims=True)
    y = xc * lax.rsqrt(var + 1e-5) * g_ref[...] + b_ref[...]
    out_ref[...] = jnp.maximum(y, 0.0)


def kernel(curr_h, next_h, curr_inc, edge_index, W_conv, b_conv,
           W_fusion, b_fusion, conv_w, topDown_w, ln_gamma, ln_beta):
    N, D = curr_h.shape
    M = next_h.shape[0]
    E = edge_index.shape[1]
    NA = -(-(N + 1) // (NS * 128)) * (NS * 128)
    f32 = jnp.float32

    src = edge_index[0].astype(jnp.int32)
    dst = edge_index[1].astype(jnp.int32)

    ET = -(-E // (NS * 128)) * 128
    pad_d = NS * ET - E
    idx_d = jnp.concatenate([
        jnp.concatenate([src, jnp.full((pad_d,), N, jnp.int32)]),
        jnp.concatenate([dst, jnp.full((pad_d,), N, jnp.int32)]),
    ])
    zeros_row = jnp.zeros((NA,), f32)
    deg32 = _make_degree_kernel(ET, NA)(idx_d, zeros_row).reshape(NW, NA)
    deg_out_t = deg32[:NS, :N].T
    deg_in_t = deg32[NS:, :N].T

    P = pl.pallas_call(
        _proj_body,
        out_shape=jax.ShapeDtypeStruct((M, D), f32),
    )(next_h, W_fusion, topDown_w.reshape(1, D))

    BN, BM = 400, M
    kb = M // BM
    zs = pl.pallas_call(
        functools.partial(_zs_body, kb=kb),
        grid=(N // BN, kb),
        in_specs=[
            pl.BlockSpec((BN, BM), lambda i, k: (i, k)),
            pl.BlockSpec((BM, D), lambda i, k: (k, 0)),
            pl.BlockSpec((BN, D), lambda i, k: (i, 0)),
            pl.BlockSpec((D, D), lambda i, k: (0, 0)),
            pl.BlockSpec((1, D), lambda i, k: (0, 0)),
            pl.BlockSpec((BN, NS), lambda i, k: (i, 0)),
        ],
        out_specs=pl.BlockSpec((BN, D), lambda i, k: (i, 0)),
        out_shape=jax.ShapeDtypeStruct((N, D), f32),
        compiler_params=pltpu.CompilerParams(
            dimension_semantics=("parallel", "arbitrary")),
    )(curr_inc, P, curr_h, W_conv, conv_w.reshape(1, D), deg_out_t)

    zs_pad = jnp.concatenate([zs, jnp.zeros((NA - N, D), f32)], axis=0)

    KM = 2 * -(-E // (NW * CE * 2))
    pad_m = NW * CE * KM - E
    src_m = jnp.concatenate([src, jnp.zeros((pad_m,), jnp.int32)])
    dst_m = jnp.concatenate([dst, jnp.full((pad_m,), N, jnp.int32)])
    pair = jnp.stack([src_m.reshape(NW, KM, CE),
                      dst_m.reshape(NW, KM, CE)], axis=2).reshape(-1, 2, CE)
    zeros_big = jnp.zeros((NA, D), f32)
    partials = _make_edge_kernel(KM, NA, D)(pair, zs_pad, zeros_big)

    BF = 400
    out = pl.pallas_call(
        _final_body,
        grid=(N // BF,),
        in_specs=[
            pl.BlockSpec((BF, D), lambda i: (i, 0)),
            pl.BlockSpec((BF, D), lambda i: (i, 0)),
            pl.BlockSpec((BF, NS), lambda i: (i, 0)),
        ] + [pl.BlockSpec((1, D), lambda i: (0, 0))] * 6,
        out_specs=pl.BlockSpec((BF, D), lambda i: (i, 0)),
        out_shape=jax.ShapeDtypeStruct((N, D), f32),
    )(partials[0, :N], partials[1, :N], deg_in_t,
      b_conv.reshape(1, D), conv_w.reshape(1, D),
      b_fusion.reshape(1, D), topDown_w.reshape(1, D),
      ln_gamma.reshape(1, D), ln_beta.reshape(1, D))
    return out


# speedup vs baseline: 1.1138x; 1.1138x over previous
import functools

import jax
import jax.numpy as jnp
from jax import lax
from jax.experimental import pallas as pl
from jax.experimental.pallas import tpu as pltpu
from jax.experimental.pallas import tpu_sc as plsc

NC = 2
NS = 16
NW = NC * NS
CE = 128


def _make_degree_kernel(ET, NA):
    mesh = plsc.VectorSubcoreMesh(core_axis_name="c", subcore_axis_name="s")

    @functools.partial(
        pl.kernel,
        out_type=jax.ShapeDtypeStruct((NW * NA,), jnp.float32),
        mesh=mesh,
        compiler_params=pltpu.CompilerParams(needs_layout_passes=False),
        scratch_types=[
            pltpu.VMEM((ET,), jnp.int32),
            pltpu.VMEM((NA,), jnp.float32),
        ],
    )
    def deg_kernel(idx_hbm, zeros_hbm, out_hbm, idx_v, local):
        cid = lax.axis_index("c")
        sid = lax.axis_index("s")
        wid = cid * NS + sid
        pltpu.sync_copy(zeros_hbm, local)
        pltpu.sync_copy(idx_hbm.at[pl.ds(wid * ET, ET)], idx_v)
        ones16 = jnp.ones((16,), jnp.float32)

        def body(i, carry):
            idx16 = idx_v[pl.ds(i * 16, 16)]
            plsc.addupdate_scatter(local, [idx16], ones16)
            return carry

        lax.fori_loop(0, ET // 16, body, 0)
        pltpu.sync_copy(local, out_hbm.at[pl.ds(wid * NA, NA)])

    return deg_kernel


def _make_edge_kernel(KM, NA, D):
    mesh = plsc.VectorSubcoreMesh(core_axis_name="c", subcore_axis_name="s")
    rows = NA // NS
    assert KM % 2 == 0

    @functools.partial(
        pl.kernel,
        out_type=jax.ShapeDtypeStruct((2, NA, D), jnp.float32),
        mesh=mesh,
        scratch_types=[
            [pltpu.VMEM((CE,), jnp.int32)] * 2,
            [pltpu.VMEM((CE,), jnp.int32)] * 2,
            [pltpu.VMEM((CE, D), jnp.float32)] * 2,
            [pltpu.SemaphoreType.DMA] * 2,
            [pltpu.SemaphoreType.DMA] * 2,
            [pltpu.SemaphoreType.DMA] * 2,
            pltpu.VMEM_SHARED((NA, D), jnp.float32),
        ],
    )
    def edge_kernel(src_hbm, dst_hbm, zs_hbm, zeros_hbm, out_hbm,
                    src_c, dst_c, bufs, ssem, dsem, gsem, acc):
        cid = lax.axis_index("c")
        sid = lax.axis_index("s")
        wid = cid * NS + sid
        r0 = sid * rows
        e0 = wid * (KM * CE)
        @pl.when(cid == 0)
        def _():
            pltpu.sync_copy(zs_hbm.at[pl.ds(r0, rows)], acc.at[pl.ds(r0, rows)])

        @pl.when(cid == 1)
        def _():
            pltpu.sync_copy(zeros_hbm.at[pl.ds(r0, rows)], acc.at[pl.ds(r0, rows)])

        for b in range(2):
            pltpu.async_copy(src_hbm.at[pl.ds(e0 + b * CE, CE)], src_c[b],
                             ssem[b])
            pltpu.async_copy(dst_hbm.at[pl.ds(e0 + b * CE, CE)], dst_c[b],
                             dsem[b])
        plsc.subcore_barrier()
        pltpu.make_async_copy(src_hbm.at[pl.ds(0, CE)], src_c[0],
                              ssem[0]).wait()
        pltpu.async_copy(zs_hbm.at[src_c[0]], bufs[0], gsem[0])

        def body(g, carry):
            for b in range(2):
                k = g * 2 + b
                nb = 1 - b

                @pl.when(k + 1 < KM)
                def _():
                    pltpu.make_async_copy(src_hbm.at[pl.ds(0, CE)], src_c[nb],
                                          ssem[nb]).wait()
                    pltpu.async_copy(zs_hbm.at[src_c[nb]], bufs[nb], gsem[nb])

                pltpu.make_async_copy(zs_hbm.at[src_c[b]], bufs[b],
                                      gsem[b]).wait()

                @pl.when(k + 2 < KM)
                def _():
                    pltpu.async_copy(src_hbm.at[pl.ds(e0 + (k + 2) * CE, CE)],
                                     src_c[b], ssem[b])

                pltpu.make_async_copy(dst_hbm.at[pl.ds(0, CE)], dst_c[b],
                                      dsem[b]).wait()
                pltpu.sync_copy(bufs[b], acc.at[dst_c[b]], add=True)

                @pl.when(k + 2 < KM)
                def _():
                    pltpu.async_copy(dst_hbm.at[pl.ds(e0 + (k + 2) * CE, CE)],
                                     dst_c[b], dsem[b])

            return carry

        lax.fori_loop(0, KM // 2, body, 0)
        plsc.subcore_barrier()
        pltpu.sync_copy(acc.at[pl.ds(r0, rows)], out_hbm.at[cid, pl.ds(r0, rows)])

    return edge_kernel


def _proj_body(next_h_ref, w_ref, tw_ref, out_ref):
    w2 = w_ref[...] * tw_ref[...]
    out_ref[...] = jnp.dot(next_h_ref[...], w2,
                           preferred_element_type=jnp.float32,
                           precision=lax.Precision.HIGHEST)


def _zs_body(inc_ref, p_ref, h_ref, wc_ref, cw_ref, deg_ref, out_ref, *, kb):
    k = pl.program_id(1)

    @pl.when(k == 0)
    def _():
        out_ref[...] = jnp.zeros_like(out_ref)

    out_ref[...] += jnp.dot(inc_ref[...], p_ref[...],
                            preferred_element_type=jnp.float32,
                            precision=lax.Precision.HIGHEST)

    @pl.when(k == kb - 1)
    def _():
        w1 = wc_ref[...] * cw_ref[...]
        z = out_ref[...] + jnp.dot(h_ref[...], w1,
                                   preferred_element_type=jnp.float32,
                                   precision=lax.Precision.HIGHEST)
        deg = jnp.sum(deg_ref[...], axis=-1, keepdims=True)
        out_ref[...] = z * lax.rsqrt(deg + 1.0)


def _final_body(p0_ref, p1_ref, deg_ref, bc_ref, cw_ref, bf_ref, tw_ref,
                g_ref, b_ref, out_ref):
    bias = bc_ref[...] * cw_ref[...] + bf_ref[...] * tw_ref[...]
    deg = jnp.sum(deg_ref[...], axis=-1, keepdims=True)
    x = (p0_ref[...] + p1_ref[...]) * lax.rsqrt(deg + 1.0) + bias
    mu = jnp.mean(x, axis=-1, keepdims=True)
    xc = x - mu
    var = jnp.mean(xc * xc, axis=-1, keepdims=True)
    y = xc * lax.rsqrt(var + 1e-5) * g_ref[...] + b_ref[...]
    out_ref[...] = jnp.maximum(y, 0.0)


def kernel(curr_h, next_h, curr_inc, edge_index, W_conv, b_conv,
           W_fusion, b_fusion, conv_w, topDown_w, ln_gamma, ln_beta):
    N, D = curr_h.shape
    M = next_h.shape[0]
    E = edge_index.shape[1]
    NA = -(-(N + 1) // (NS * 128)) * (NS * 128)
    f32 = jnp.float32

    src = edge_index[0].astype(jnp.int32)
    dst = edge_index[1].astype(jnp.int32)

    ET = -(-E // (NS * 128)) * 128
    pad_d = NS * ET - E
    idx_d = jnp.concatenate([
        jnp.concatenate([src, jnp.full((pad_d,), N, jnp.int32)]),
        jnp.concatenate([dst, jnp.full((pad_d,), N, jnp.int32)]),
    ])
    zeros_row = jnp.zeros((NA,), f32)
    deg32 = _make_degree_kernel(ET, NA)(idx_d, zeros_row).reshape(NW, NA)
    deg_out_t = deg32[:NS, :N].T
    deg_in_t = deg32[NS:, :N].T

    P = pl.pallas_call(
        _proj_body,
        out_shape=jax.ShapeDtypeStruct((M, D), f32),
    )(next_h, W_fusion, topDown_w.reshape(1, D))

    BN, BM = 400, M
    kb = M // BM
    zs = pl.pallas_call(
        functools.partial(_zs_body, kb=kb),
        grid=(N // BN, kb),
        in_specs=[
            pl.BlockSpec((BN, BM), lambda i, k: (i, k)),
            pl.BlockSpec((BM, D), lambda i, k: (k, 0)),
            pl.BlockSpec((BN, D), lambda i, k: (i, 0)),
            pl.BlockSpec((D, D), lambda i, k: (0, 0)),
            pl.BlockSpec((1, D), lambda i, k: (0, 0)),
            pl.BlockSpec((BN, NS), lambda i, k: (i, 0)),
        ],
        out_specs=pl.BlockSpec((BN, D), lambda i, k: (i, 0)),
        out_shape=jax.ShapeDtypeStruct((N, D), f32),
        compiler_params=pltpu.CompilerParams(
            dimension_semantics=("parallel", "arbitrary")),
    )(curr_inc, P, curr_h, W_conv, conv_w.reshape(1, D), deg_out_t)

    zs_pad = jnp.concatenate([zs, jnp.zeros((NA - N, D), f32)], axis=0)

    KM = 2 * -(-E // (NW * CE * 2))
    pad_m = NW * CE * KM - E
    src_m = jnp.concatenate([src, jnp.zeros((pad_m,), jnp.int32)])
    dst_m = jnp.concatenate([dst, jnp.full((pad_m,), N, jnp.int32)])
    zeros_big = jnp.zeros((NA, D), f32)
    partials = _make_edge_kernel(KM, NA, D)(src_m, dst_m, zs_pad, zeros_big)

    BF = 400
    out = pl.pallas_call(
        _final_body,
        grid=(N // BF,),
        in_specs=[
            pl.BlockSpec((BF, D), lambda i: (i, 0)),
            pl.BlockSpec((BF, D), lambda i: (i, 0)),
            pl.BlockSpec((BF, NS), lambda i: (i, 0)),
        ] + [pl.BlockSpec((1, D), lambda i: (0, 0))] * 6,
        out_specs=pl.BlockSpec((BF, D), lambda i: (i, 0)),
        out_shape=jax.ShapeDtypeStruct((N, D), f32),
    )(partials[0, :N], partials[1, :N], deg_in_t,
      b_conv.reshape(1, D), conv_w.reshape(1, D),
      b_fusion.reshape(1, D), topDown_w.reshape(1, D),
      ln_gamma.reshape(1, D), ln_beta.reshape(1, D))
    return out


# speedup vs baseline: 1.1421x; 1.0254x over previous
import functools

import jax
import jax.numpy as jnp
from jax import lax
from jax.experimental import pallas as pl
from jax.experimental.pallas import tpu as pltpu
from jax.experimental.pallas import tpu_sc as plsc

NC = 2
NS = 16
NW = NC * NS
CE = 128


def _make_degree_kernel(ET, NA):
    mesh = plsc.VectorSubcoreMesh(core_axis_name="c", subcore_axis_name="s")

    @functools.partial(
        pl.kernel,
        out_type=jax.ShapeDtypeStruct((NW * NA,), jnp.float32),
        mesh=mesh,
        compiler_params=pltpu.CompilerParams(needs_layout_passes=False),
        scratch_types=[
            pltpu.VMEM((ET,), jnp.int32),
            pltpu.VMEM((NA,), jnp.float32),
        ],
    )
    def deg_kernel(idx_hbm, zeros_hbm, out_hbm, idx_v, local):
        cid = lax.axis_index("c")
        sid = lax.axis_index("s")
        wid = cid * NS + sid
        pltpu.sync_copy(zeros_hbm, local)
        pltpu.sync_copy(idx_hbm.at[pl.ds(wid * ET, ET)], idx_v)
        ones16 = jnp.ones((16,), jnp.float32)

        def body(i, carry):
            for j in range(8):
                idx16 = idx_v[pl.ds(i * 128 + j * 16, 16)]
                plsc.addupdate_scatter(local, [idx16], ones16)
            return carry

        lax.fori_loop(0, ET // 128, body, 0)
        pltpu.sync_copy(local, out_hbm.at[pl.ds(wid * NA, NA)])

    return deg_kernel


def _make_edge_kernel(KM, NA, D):
    mesh = plsc.VectorSubcoreMesh(core_axis_name="c", subcore_axis_name="s")
    rows = NA // NS
    assert KM % 2 == 0

    @functools.partial(
        pl.kernel,
        out_type=jax.ShapeDtypeStruct((2, NA, D), jnp.float32),
        mesh=mesh,
        scratch_types=[
            [pltpu.VMEM((CE,), jnp.int32)] * 2,
            [pltpu.VMEM((CE,), jnp.int32)] * 2,
            [pltpu.VMEM((CE, D), jnp.float32)] * 2,
            [pltpu.SemaphoreType.DMA] * 2,
            [pltpu.SemaphoreType.DMA] * 2,
            [pltpu.SemaphoreType.DMA] * 2,
            pltpu.VMEM_SHARED((NA, D), jnp.float32),
        ],
    )
    def edge_kernel(src_hbm, dst_hbm, zs_hbm, zeros_hbm, out_hbm,
                    src_c, dst_c, bufs, ssem, dsem, gsem, acc):
        cid = lax.axis_index("c")
        sid = lax.axis_index("s")
        wid = cid * NS + sid
        r0 = sid * rows
        e0 = wid * (KM * CE)
        @pl.when(cid == 0)
        def _():
            pltpu.sync_copy(zs_hbm.at[pl.ds(r0, rows)], acc.at[pl.ds(r0, rows)])

        @pl.when(cid == 1)
        def _():
            pltpu.sync_copy(zeros_hbm.at[pl.ds(r0, rows)], acc.at[pl.ds(r0, rows)])

        for b in range(2):
            pltpu.async_copy(src_hbm.at[pl.ds(e0 + b * CE, CE)], src_c[b],
                             ssem[b])
            pltpu.async_copy(dst_hbm.at[pl.ds(e0 + b * CE, CE)], dst_c[b],
                             dsem[b])
        plsc.subcore_barrier()
        pltpu.make_async_copy(src_hbm.at[pl.ds(0, CE)], src_c[0],
                              ssem[0]).wait()
        pltpu.async_copy(zs_hbm.at[src_c[0]], bufs[0], gsem[0])

        def body(g, carry):
            for b in range(2):
                k = g * 2 + b
                nb = 1 - b

                @pl.when(k + 1 < KM)
                def _():
                    pltpu.make_async_copy(src_hbm.at[pl.ds(0, CE)], src_c[nb],
                                          ssem[nb]).wait()
                    pltpu.async_copy(zs_hbm.at[src_c[nb]], bufs[nb], gsem[nb])

                pltpu.make_async_copy(zs_hbm.at[src_c[b]], bufs[b],
                                      gsem[b]).wait()

                @pl.when(k + 2 < KM)
                def _():
                    pltpu.async_copy(src_hbm.at[pl.ds(e0 + (k + 2) * CE, CE)],
                                     src_c[b], ssem[b])

                pltpu.make_async_copy(dst_hbm.at[pl.ds(0, CE)], dst_c[b],
                                      dsem[b]).wait()
                pltpu.sync_copy(bufs[b], acc.at[dst_c[b]], add=True)

                @pl.when(k + 2 < KM)
                def _():
                    pltpu.async_copy(dst_hbm.at[pl.ds(e0 + (k + 2) * CE, CE)],
                                     dst_c[b], dsem[b])

            return carry

        lax.fori_loop(0, KM // 2, body, 0)
        plsc.subcore_barrier()
        pltpu.sync_copy(acc.at[pl.ds(r0, rows)], out_hbm.at[cid, pl.ds(r0, rows)])

    return edge_kernel


def _proj_body(next_h_ref, w_ref, tw_ref, out_ref):
    w2 = w_ref[...] * tw_ref[...]
    out_ref[...] = jnp.dot(next_h_ref[...], w2,
                           preferred_element_type=jnp.float32,
                           precision=lax.Precision.HIGHEST)


def _z_body(inc_ref, p_ref, h_ref, wc_ref, cw_ref, out_ref):
    w1 = wc_ref[...] * cw_ref[...]
    out_ref[...] = (
        jnp.dot(inc_ref[...], p_ref[...],
                preferred_element_type=jnp.float32,
                precision=lax.Precision.HIGHEST)
        + jnp.dot(h_ref[...], w1,
                  preferred_element_type=jnp.float32,
                  precision=lax.Precision.HIGHEST))


def _scale_body(z_ref, deg_ref, out_ref):
    deg = jnp.sum(deg_ref[...], axis=-1, keepdims=True)
    out_ref[...] = z_ref[...] * lax.rsqrt(deg + 1.0)


def _final_body(p0_ref, p1_ref, deg_ref, bc_ref, cw_ref, bf_ref, tw_ref,
                g_ref, b_ref, out_ref):
    bias = bc_ref[...] * cw_ref[...] + bf_ref[...] * tw_ref[...]
    deg = jnp.sum(deg_ref[...], axis=-1, keepdims=True)
    x = (p0_ref[0] + p1_ref[0]) * lax.rsqrt(deg + 1.0) + bias
    mu = jnp.mean(x, axis=-1, keepdims=True)
    xc = x - mu
    var = jnp.mean(xc * xc, axis=-1, keepdims=True)
    y = xc * lax.rsqrt(var + 1e-5) * g_ref[...] + b_ref[...]
    out_ref[...] = jnp.maximum(y, 0.0)


def kernel(curr_h, next_h, curr_inc, edge_index, W_conv, b_conv,
           W_fusion, b_fusion, conv_w, topDown_w, ln_gamma, ln_beta):
    N, D = curr_h.shape
    M = next_h.shape[0]
    E = edge_index.shape[1]
    NA = -(-(N + 1) // (NS * 128)) * (NS * 128)
    f32 = jnp.float32

    src = edge_index[0].astype(jnp.int32)
    dst = edge_index[1].astype(jnp.int32)

    ET = -(-E // (NS * 128)) * 128
    pad_d = NS * ET - E
    idx_d = jnp.concatenate([
        jnp.concatenate([src, jnp.full((pad_d,), N, jnp.int32)]),
        jnp.concatenate([dst, jnp.full((pad_d,), N, jnp.int32)]),
    ])
    zeros_row = jnp.zeros((NA,), f32)
    deg32 = _make_degree_kernel(ET, NA)(idx_d, zeros_row).reshape(NW, NA)
    deg_out_t = deg32[:NS, :N].T
    deg_in_t = deg32[NS:, :N].T

    P = pl.pallas_call(
        _proj_body,
        out_shape=jax.ShapeDtypeStruct((M, D), f32),
    )(next_h, W_fusion, topDown_w.reshape(1, D))

    BN = 400
    z = pl.pallas_call(
        _z_body,
        grid=(N // BN,),
        in_specs=[
            pl.BlockSpec((BN, M), lambda i: (i, 0)),
            pl.BlockSpec((M, D), lambda i: (0, 0)),
            pl.BlockSpec((BN, D), lambda i: (i, 0)),
            pl.BlockSpec((D, D), lambda i: (0, 0)),
            pl.BlockSpec((1, D), lambda i: (0, 0)),
        ],
        out_specs=pl.BlockSpec((BN, D), lambda i: (i, 0)),
        out_shape=jax.ShapeDtypeStruct((N, D), f32),
        compiler_params=pltpu.CompilerParams(
            dimension_semantics=("parallel",)),
    )(curr_inc, P, curr_h, W_conv, conv_w.reshape(1, D))

    BS = 512
    zs_pad = pl.pallas_call(
        _scale_body,
        grid=(NA // BS,),
        in_specs=[
            pl.BlockSpec((BS, D), lambda i: (i, 0)),
            pl.BlockSpec((BS, NS), lambda i: (i, 0)),
        ],
        out_specs=pl.BlockSpec((BS, D), lambda i: (i, 0)),
        out_shape=jax.ShapeDtypeStruct((NA, D), f32),
    )(z, deg_out_t)

    KM = 2 * -(-E // (NW * CE * 2))
    pad_m = NW * CE * KM - E
    src_m = jnp.concatenate([src, jnp.zeros((pad_m,), jnp.int32)])
    dst_m = jnp.concatenate([dst, jnp.full((pad_m,), N, jnp.int32)])
    zeros_big = jnp.zeros((NA, D), f32)
    partials = _make_edge_kernel(KM, NA, D)(src_m, dst_m, zs_pad, zeros_big)

    BF = 400
    out = pl.pallas_call(
        _final_body,
        grid=(N // BF,),
        in_specs=[
            pl.BlockSpec((1, BF, D), lambda i: (0, i, 0)),
            pl.BlockSpec((1, BF, D), lambda i: (1, i, 0)),
            pl.BlockSpec((BF, NS), lambda i: (i, 0)),
        ] + [pl.BlockSpec((1, D), lambda i: (0, 0))] * 6,
        out_specs=pl.BlockSpec((BF, D), lambda i: (i, 0)),
        out_shape=jax.ShapeDtypeStruct((N, D), f32),
    )(partials, partials, deg_in_t,
      b_conv.reshape(1, D), conv_w.reshape(1, D),
      b_fusion.reshape(1, D), topDown_w.reshape(1, D),
      ln_gamma.reshape(1, D), ln_beta.reshape(1, D))
    return out


# speedup vs baseline: 1.2835x; 1.1238x over previous
import functools

import jax
import jax.numpy as jnp
from jax import lax
from jax.experimental import pallas as pl
from jax.experimental.pallas import tpu as pltpu
from jax.experimental.pallas import tpu_sc as plsc

NC = 2
NS = 16
NW = NC * NS
C = 128


def _make_degree_kernel(KD, NA):
    mesh = plsc.VectorSubcoreMesh(core_axis_name="c", subcore_axis_name="s")
    rows = NA // NS

    @functools.partial(
        pl.kernel,
        out_type=jax.ShapeDtypeStruct((2 * NA,), jnp.float32),
        mesh=mesh,
        scratch_types=[
            pltpu.VMEM((KD, C), jnp.int32),
            pltpu.VMEM((C,), jnp.float32),
            pltpu.VMEM_SHARED((NA,), jnp.float32),
        ],
    )
    def deg_kernel(idx_hbm, zeros_hbm, out_hbm, idx_v, ones_v, acc):
        cid = lax.axis_index("c")
        sid = lax.axis_index("s")
        r0 = sid * rows
        for i in range(C // 16):
            ones_v[pl.ds(i * 16, 16)] = jnp.ones((16,), jnp.float32)
        pltpu.sync_copy(zeros_hbm.at[pl.ds(r0, rows)], acc.at[pl.ds(r0, rows)])
        pltpu.sync_copy(idx_hbm.at[cid, sid], idx_v)
        plsc.subcore_barrier()

        def body(k, carry):
            pltpu.sync_copy(ones_v, acc.at[idx_v.at[k]], add=True)
            return carry

        lax.fori_loop(0, KD, body, 0)
        plsc.subcore_barrier()
        pltpu.sync_copy(acc.at[pl.ds(r0, rows)],
                        out_hbm.at[pl.ds(cid * NA + r0, rows)])

    return deg_kernel


def _make_edge_kernel(KM, NA, D):
    mesh = plsc.VectorSubcoreMesh(core_axis_name="c", subcore_axis_name="s")
    rows = NA // NS

    @functools.partial(
        pl.kernel,
        out_type=jax.ShapeDtypeStruct((2, NA, D), jnp.float32),
        mesh=mesh,
        scratch_types=[
            pltpu.VMEM((KM, C), jnp.int32),
            pltpu.VMEM((KM, C), jnp.int32),
            pltpu.VMEM((C, D), jnp.float32),
            pltpu.VMEM_SHARED((NA, D), jnp.float32),
            pltpu.SemaphoreType.DMA,
        ],
    )
    def edge_kernel(src_hbm, dst_hbm, zs_hbm, zeros_hbm, out_hbm,
                    src_v, dst_v, rows_v, acc, gsem):
        cid = lax.axis_index("c")
        sid = lax.axis_index("s")
        wid = cid * NS + sid
        r0 = sid * rows
        @pl.when(cid == 0)
        def _():
            pltpu.sync_copy(zs_hbm.at[pl.ds(r0, rows)], acc.at[pl.ds(r0, rows)])

        @pl.when(cid == 1)
        def _():
            pltpu.sync_copy(zeros_hbm.at[pl.ds(r0, rows)], acc.at[pl.ds(r0, rows)])

        pltpu.sync_copy(src_hbm.at[wid], src_v)
        pltpu.sync_copy(dst_hbm.at[wid], dst_v)
        plsc.subcore_barrier()

        def body(k, carry):
            pltpu.async_copy(zs_hbm.at[src_v.at[k]], rows_v, gsem).wait()
            pltpu.sync_copy(rows_v, acc.at[dst_v.at[k]], add=True)
            return carry

        lax.fori_loop(0, KM, body, 0)
        plsc.subcore_barrier()
        pltpu.sync_copy(acc.at[pl.ds(r0, rows)], out_hbm.at[cid, pl.ds(r0, rows)])

    return edge_kernel


def _proj_body(next_h_ref, w_ref, tw_ref, out_ref):
    w2 = w_ref[...] * tw_ref[...]
    out_ref[...] = jnp.dot(next_h_ref[...], w2,
                           preferred_element_type=jnp.float32,
                           precision=lax.Precision.HIGHEST)


def _zs_body(inc_ref, p_ref, h_ref, wc_ref, cw_ref, deg_ref, out_ref):
    w1 = wc_ref[...] * cw_ref[...]
    z = (jnp.dot(inc_ref[...], p_ref[...],
                 preferred_element_type=jnp.float32,
                 precision=lax.Precision.HIGHEST)
         + jnp.dot(h_ref[...], w1,
                   preferred_element_type=jnp.float32,
                   precision=lax.Precision.HIGHEST))
    out_ref[...] = z * lax.rsqrt(deg_ref[...] + 1.0)


def _final_body(p0_ref, p1_ref, deg_ref, bc_ref, cw_ref, bf_ref, tw_ref,
                g_ref, b_ref, out_ref):
    bias = bc_ref[...] * cw_ref[...] + bf_ref[...] * tw_ref[...]
    x = (p0_ref[0] + p1_ref[0]) * lax.rsqrt(deg_ref[...] + 1.0) + bias
    mu = jnp.mean(x, axis=-1, keepdims=True)
    xc = x - mu
    var = jnp.mean(xc * xc, axis=-1, keepdims=True)
    y = xc * lax.rsqrt(var + 1e-5) * g_ref[...] + b_ref[...]
    out_ref[...] = jnp.maximum(y, 0.0)


def kernel(curr_h, next_h, curr_inc, edge_index, W_conv, b_conv,
           W_fusion, b_fusion, conv_w, topDown_w, ln_gamma, ln_beta):
    N, D = curr_h.shape
    M = next_h.shape[0]
    E = edge_index.shape[1]
    NA = -(-(N + 1) // (NS * 128)) * (NS * 128)
    f32 = jnp.float32

    src = edge_index[0].astype(jnp.int32)
    dst = edge_index[1].astype(jnp.int32)

    KD = -(-E // (NS * C))
    pad_d = NS * C * KD - E
    idx_d = jnp.stack([
        jnp.concatenate([src, jnp.full((pad_d,), N, jnp.int32)]),
        jnp.concatenate([dst, jnp.full((pad_d,), N, jnp.int32)]),
    ]).reshape(2, NS, KD, C)
    zeros_row = jnp.zeros((NA,), f32)
    deg = _make_degree_kernel(KD, NA)(idx_d, zeros_row)
    deg_out_col = deg[:NA].reshape(NA, 1)
    deg_in_col = deg[NA:].reshape(NA, 1)

    P = pl.pallas_call(
        _proj_body,
        out_shape=jax.ShapeDtypeStruct((M, D), f32),
    )(next_h, W_fusion, topDown_w.reshape(1, D))

    BN = 512
    zs_pad = pl.pallas_call(
        _zs_body,
        grid=(NA // BN,),
        in_specs=[
            pl.BlockSpec((BN, M), lambda i: (i, 0)),
            pl.BlockSpec((M, D), lambda i: (0, 0)),
            pl.BlockSpec((BN, D), lambda i: (i, 0)),
            pl.BlockSpec((D, D), lambda i: (0, 0)),
            pl.BlockSpec((1, D), lambda i: (0, 0)),
            pl.BlockSpec((BN, 1), lambda i: (i, 0)),
        ],
        out_specs=pl.BlockSpec((BN, D), lambda i: (i, 0)),
        out_shape=jax.ShapeDtypeStruct((NA, D), f32),
        compiler_params=pltpu.CompilerParams(
            dimension_semantics=("parallel",)),
    )(curr_inc, P, curr_h, W_conv, conv_w.reshape(1, D), deg_out_col)

    KM = -(-E // (NW * C))
    pad_m = NW * C * KM - E
    src_m = jnp.concatenate([src, jnp.zeros((pad_m,), jnp.int32)])
    dst_m = jnp.concatenate([dst, jnp.full((pad_m,), N, jnp.int32)])
    src_m = src_m.reshape(NW, KM, C)
    dst_m = dst_m.reshape(NW, KM, C)
    zeros_big = jnp.zeros((NA, D), f32)
    partials = _make_edge_kernel(KM, NA, D)(src_m, dst_m, zs_pad, zeros_big)

    BF = 400
    out = pl.pallas_call(
        _final_body,
        grid=(N // BF,),
        in_specs=[
            pl.BlockSpec((1, BF, D), lambda i: (0, i, 0)),
            pl.BlockSpec((1, BF, D), lambda i: (1, i, 0)),
            pl.BlockSpec((BF, 1), lambda i: (i, 0)),
        ] + [pl.BlockSpec((1, D), lambda i: (0, 0))] * 6,
        out_specs=pl.BlockSpec((BF, D), lambda i: (i, 0)),
        out_shape=jax.ShapeDtypeStruct((N, D), f32),
    )(partials, partials, deg_in_col,
      b_conv.reshape(1, D), conv_w.reshape(1, D),
      b_fusion.reshape(1, D), topDown_w.reshape(1, D),
      ln_gamma.reshape(1, D), ln_beta.reshape(1, D))
    return out
